# baseline (speedup 1.0000x reference)
import functools

import jax
import jax.numpy as jnp
from jax import lax
from jax.experimental import pallas as pl
from jax.experimental.pallas import tpu as pltpu
from jax.experimental.pallas import tpu_sc as plsc

_N = 1_600_000
_G = 10_000
_STEP = 0.01
_NC = 2
_NS = 16
_NW = _NC * _NS
_NT = _N // _NW
_CH = 10_000
_NCHUNK = _NT // _CH
_L = 16
_UNROLL = 25
_GP = 10_240
_GC = _GP // _NS


def _hist_body(loss_hbm, ids_hbm, out_sums, out_counts,
               sums, counts, ids0, ids1, loss0, loss1, blk, blk2, red, red2,
               shs, shc, sem0, sem1):
    cid = lax.axis_index("c")
    sid = lax.axis_index("s")
    wid = cid * _NS + sid

    zero = jnp.zeros((_L,), jnp.float32)
    ones = jnp.ones((_L,), jnp.float32)

    def zbody(i, _):
        for u in range(8):
            d = pl.ds((i * 8 + u) * _L, _L)
            sums[d] = zero
            counts[d] = zero
        return 0
    lax.fori_loop(0, _GP // (_L * 8), zbody, 0)

    base = wid * _NT
    bufs = ((ids0, loss0, sem0), (ids1, loss1, sem1))

    def start(c, b):
        ib, lb, sem = bufs[b]
        off = base + c * _CH
        pltpu.make_async_copy(ids_hbm.at[pl.ds(off, _CH)], ib, sem).start()
        pltpu.make_async_copy(loss_hbm.at[pl.ds(off, _CH)], lb, sem).start()

    def wait(b):
        ib, lb, sem = bufs[b]
        pltpu.make_async_copy(ids_hbm.at[pl.ds(0, _CH)], ib, sem).wait()
        pltpu.make_async_copy(loss_hbm.at[pl.ds(0, _CH)], lb, sem).wait()

    def process(b):
        ib, lb, _ = bufs[b]

        def vbody(i, _):
            for u in range(_UNROLL):
                d = pl.ds((i * _UNROLL + u) * _L, _L)
                idx = ib[d]
                vals = lb[d]
                plsc.addupdate_scatter(sums, [idx], vals)
                plsc.addupdate_scatter(counts, [idx], ones)
            return 0
        lax.fori_loop(0, _CH // (_L * _UNROLL), vbody, 0)

    start(0, 0)
    start(1, 1)

    def chunk_body(k, _):
        c = k * 2
        wait(0)
        process(0)
        start(c + 2, 0)
        wait(1)
        process(1)

        @pl.when(c + 3 < _NCHUNK)
        def _():
            start(c + 3, 1)
        return 0
    lax.fori_loop(0, _NCHUNK // 2, chunk_body, 0)
    wait(0)
    process(0)

    p0 = pltpu.make_async_copy(sums, shs.at[sid], sem0)
    p1 = pltpu.make_async_copy(counts, shc.at[sid], sem1)
    p0.start()
    p1.start()
    p0.wait()
    p1.wait()
    plsc.subcore_barrier()

    gbase = sid * _GC
    g0 = pltpu.make_async_copy(shs.at[:, pl.ds(gbase, _GC)], blk, sem0)
    g1 = pltpu.make_async_copy(shc.at[:, pl.ds(gbase, _GC)], blk2, sem1)
    g0.start()
    g1.start()

    def reduce_block(b, r):
        def rbody(j, _):
            acc = b[0, pl.ds(j * _L, _L)]
            for rr in range(1, _NS):
                acc = acc + b[rr, pl.ds(j * _L, _L)]
            r[pl.ds(j * _L, _L)] = acc
            return 0
        lax.fori_loop(0, _GC // _L, rbody, 0)

    g0.wait()
    reduce_block(blk, red)
    w0 = pltpu.make_async_copy(red, out_sums.at[cid, pl.ds(gbase, _GC)], sem0)
    w0.start()
    g1.wait()
    reduce_block(blk2, red2)
    w1 = pltpu.make_async_copy(red2, out_counts.at[cid, pl.ds(gbase, _GC)], sem1)
    w1.start()
    w0.wait()
    w1.wait()


def _finish_tc(ps_ref, pc_ref, w_ref, out_ref):
    s = ps_ref[0] + ps_ref[1]
    c = pc_ref[0] + pc_ref[1]
    gl = s / jnp.maximum(c, 1.0)
    u = w_ref[...] * jnp.exp(_STEP * gl)
    s_tot = jnp.sum(u)
    t_tot = jnp.sum(u * gl)
    out_ref[...] = jnp.broadcast_to(t_tot / s_tot, (1, 1))


def kernel(losses, group_ids, group_weights):
    mesh = plsc.VectorSubcoreMesh(core_axis_name="c", subcore_axis_name="s")

    hist = pl.kernel(
        _hist_body,
        out_type=(
            jax.ShapeDtypeStruct((_NC, _GP), jnp.float32),
            jax.ShapeDtypeStruct((_NC, _GP), jnp.float32),
        ),
        mesh=mesh,
        compiler_params=pltpu.CompilerParams(needs_layout_passes=False),
        scratch_types=[
            pltpu.VMEM((_GP,), jnp.float32),
            pltpu.VMEM((_GP,), jnp.float32),
            pltpu.VMEM((_CH,), jnp.int32),
            pltpu.VMEM((_CH,), jnp.int32),
            pltpu.VMEM((_CH,), jnp.float32),
            pltpu.VMEM((_CH,), jnp.float32),
            pltpu.VMEM((_NS, _GC), jnp.float32),
            pltpu.VMEM((_NS, _GC), jnp.float32),
            pltpu.VMEM((_GC,), jnp.float32),
            pltpu.VMEM((_GC,), jnp.float32),
            pltpu.VMEM_SHARED((_NS, _GP), jnp.float32),
            pltpu.VMEM_SHARED((_NS, _GP), jnp.float32),
            pltpu.SemaphoreType.DMA,
            pltpu.SemaphoreType.DMA,
        ],
    )
    part_sums, part_counts = hist(losses, group_ids)

    w_pad = jnp.pad(group_weights, (0, _GP - _G))

    ps3 = part_sums.reshape(_NC, _GP // 128, 128)
    pc3 = part_counts.reshape(_NC, _GP // 128, 128)
    w2 = w_pad.reshape(_GP // 128, 128)

    out = pl.pallas_call(
        _finish_tc,
        out_shape=jax.ShapeDtypeStruct((1, 1), jnp.float32),
    )(ps3, pc3, w2)
    return out[0, 0]


# speedup vs baseline: 1.0218x; 1.0218x over previous
import functools

import jax
import jax.numpy as jnp
from jax import lax
from jax.experimental import pallas as pl
from jax.experimental.pallas import tpu as pltpu
from jax.experimental.pallas import tpu_sc as plsc

_N = 1_600_000
_G = 10_000
_STEP = 0.01
_NC = 2
_NS = 16
_NW = _NC * _NS
_NT = _N // _NW
_CH = 2_000
_NCHUNK = _NT // _CH
_L = 16
_UNROLL = 25
_GP = 10_240
_GC = _GP // _NS


def _hist_body(loss_hbm, ids_hbm, out_sums, out_counts,
               sums, counts, ids0, ids1, loss0, loss1, blk, blk2, red, red2,
               shs, shc, sem0, sem1):
    cid = lax.axis_index("c")
    sid = lax.axis_index("s")
    wid = cid * _NS + sid

    zero = jnp.zeros((_L,), jnp.float32)
    ones = jnp.ones((_L,), jnp.float32)

    def zbody(i, _):
        for u in range(8):
            d = pl.ds((i * 8 + u) * _L, _L)
            sums[d] = zero
            counts[d] = zero
        return 0
    lax.fori_loop(0, _GP // (_L * 8), zbody, 0)

    base = wid * _NT
    bufs = ((ids0, loss0, sem0), (ids1, loss1, sem1))

    def start(c, b):
        ib, lb, sem = bufs[b]
        off = base + c * _CH
        pltpu.make_async_copy(ids_hbm.at[pl.ds(off, _CH)], ib, sem).start()
        pltpu.make_async_copy(loss_hbm.at[pl.ds(off, _CH)], lb, sem).start()

    def wait(b):
        ib, lb, sem = bufs[b]
        pltpu.make_async_copy(ids_hbm.at[pl.ds(0, _CH)], ib, sem).wait()
        pltpu.make_async_copy(loss_hbm.at[pl.ds(0, _CH)], lb, sem).wait()

    def process(b):
        ib, lb, _ = bufs[b]

        def vbody(i, _):
            for u in range(_UNROLL):
                d = pl.ds((i * _UNROLL + u) * _L, _L)
                idx = ib[d]
                vals = lb[d]
                plsc.addupdate_scatter(sums, [idx], vals)
                plsc.addupdate_scatter(counts, [idx], ones)
            return 0
        lax.fori_loop(0, _CH // (_L * _UNROLL), vbody, 0)

    start(0, 0)
    start(1, 1)

    def chunk_body(k, _):
        c = k * 2
        wait(0)
        process(0)
        start(c + 2, 0)
        wait(1)
        process(1)

        @pl.when(c + 3 < _NCHUNK)
        def _():
            start(c + 3, 1)
        return 0
    lax.fori_loop(0, _NCHUNK // 2, chunk_body, 0)
    wait(0)
    process(0)

    p0 = pltpu.make_async_copy(sums, shs.at[sid], sem0)
    p1 = pltpu.make_async_copy(counts, shc.at[sid], sem1)
    p0.start()
    p1.start()
    p0.wait()
    p1.wait()
    plsc.subcore_barrier()

    gbase = sid * _GC
    g0 = pltpu.make_async_copy(shs.at[:, pl.ds(gbase, _GC)], blk, sem0)
    g1 = pltpu.make_async_copy(shc.at[:, pl.ds(gbase, _GC)], blk2, sem1)
    g0.start()
    g1.start()

    def reduce_block(b, r):
        def rbody(j, _):
            acc = b[0, pl.ds(j * _L, _L)]
            for rr in range(1, _NS):
                acc = acc + b[rr, pl.ds(j * _L, _L)]
            r[pl.ds(j * _L, _L)] = acc
            return 0
        lax.fori_loop(0, _GC // _L, rbody, 0)

    g0.wait()
    reduce_block(blk, red)
    w0 = pltpu.make_async_copy(red, out_sums.at[cid, pl.ds(gbase, _GC)], sem0)
    w0.start()
    g1.wait()
    reduce_block(blk2, red2)
    w1 = pltpu.make_async_copy(red2, out_counts.at[cid, pl.ds(gbase, _GC)], sem1)
    w1.start()
    w0.wait()
    w1.wait()


def _finish_tc(ps_ref, pc_ref, w_ref, out_ref):
    s = ps_ref[0] + ps_ref[1]
    c = pc_ref[0] + pc_ref[1]
    gl = s / jnp.maximum(c, 1.0)
    u = w_ref[...] * jnp.exp(_STEP * gl)
    s_tot = jnp.sum(u)
    t_tot = jnp.sum(u * gl)
    out_ref[...] = jnp.broadcast_to(t_tot / s_tot, (1, 1))


def kernel(losses, group_ids, group_weights):
    mesh = plsc.VectorSubcoreMesh(core_axis_name="c", subcore_axis_name="s")

    hist = pl.kernel(
        _hist_body,
        out_type=(
            jax.ShapeDtypeStruct((_NC, _GP), jnp.float32),
            jax.ShapeDtypeStruct((_NC, _GP), jnp.float32),
        ),
        mesh=mesh,
        compiler_params=pltpu.CompilerParams(needs_layout_passes=False),
        scratch_types=[
            pltpu.VMEM((_GP,), jnp.float32),
            pltpu.VMEM((_GP,), jnp.float32),
            pltpu.VMEM((_CH,), jnp.int32),
            pltpu.VMEM((_CH,), jnp.int32),
            pltpu.VMEM((_CH,), jnp.float32),
            pltpu.VMEM((_CH,), jnp.float32),
            pltpu.VMEM((_NS, _GC), jnp.float32),
            pltpu.VMEM((_NS, _GC), jnp.float32),
            pltpu.VMEM((_GC,), jnp.float32),
            pltpu.VMEM((_GC,), jnp.float32),
            pltpu.VMEM_SHARED((_NS, _GP), jnp.float32),
            pltpu.VMEM_SHARED((_NS, _GP), jnp.float32),
            pltpu.SemaphoreType.DMA,
            pltpu.SemaphoreType.DMA,
        ],
    )
    part_sums, part_counts = hist(losses, group_ids)

    w_pad = jnp.pad(group_weights, (0, _GP - _G))

    ps3 = part_sums.reshape(_NC, _GP // 128, 128)
    pc3 = part_counts.reshape(_NC, _GP // 128, 128)
    w2 = w_pad.reshape(_GP // 128, 128)

    out = pl.pallas_call(
        _finish_tc,
        out_shape=jax.ShapeDtypeStruct((1, 1), jnp.float32),
    )(ps3, pc3, w2)
    return out[0, 0]
